# baseline (speedup 1.0000x reference)
import jax
import jax.numpy as jnp
from jax.experimental import pallas as pl
from jax.experimental.pallas import tpu as pltpu

SEQ = 512
BAT = 64
HID = 768
T = 32
TT = T * T
START = 30
END = 31
BS = 8
NBLK = SEQ // BS
ROWS = BS * BAT


def _crf_body(feats_ref, tgt_ref, msk_ref, wt_ref, b_ref, e_ref, s_ref,
              out_ref, sc_scr, part_ref, tg_ref):
    k = pl.program_id(0)
    p = jax.lax.rem(k, 2)

    @pl.when(k < NBLK)
    def _():
        fb = feats_ref[...].astype(jnp.bfloat16)
        sc_scr[pl.ds(p * ROWS, ROWS), :] = (
            jnp.dot(fb, wt_ref[...], preferred_element_type=jnp.float32)
            + b_ref[...])

    @pl.when(k > 0)
    def _():
        q = 1 - p
        lane = jax.lax.broadcasted_iota(jnp.int32, (BAT, TT), 1)
        tgt2 = tgt_ref[0]
        msk2 = msk_ref[0]
        part = part_ref[...]
        tg = jnp.where(k == 1, 0.0, tg_ref[0, 0])
        for i in range(BS):
            sc = sc_scr[pl.ds(q * ROWS + i * BAT, BAT), :]
            tcol = jax.lax.slice(tgt2, (0, i), (BAT, i + 1))
            mcol = jax.lax.slice(msk2, (0, i), (BAT, i + 1))
            tg = tg + jnp.sum(jnp.where((lane == tcol) & (mcol > 0.0), sc, 0.0))
            pmax = jnp.max(part, axis=1, keepdims=True)
            pexp = jnp.dot(part - pmax, e_ref[...],
                           preferred_element_type=jnp.float32)
            cur = sc + pexp
            mrow = jnp.max(cur, axis=1, keepdims=True)
            ex = jnp.exp(cur - mrow)
            ssum = jnp.dot(ex, s_ref[...], preferred_element_type=jnp.float32)
            rec = jnp.log(ssum) + (mrow + pmax)
            newpart = jnp.where(mcol > 0.0, rec, part)
            if i == 0:
                init = jax.lax.slice(sc, (0, START * T), (BAT, START * T + T))
                newpart = jnp.where(k == 1, init, newpart)
            part = newpart
        part_ref[...] = part
        tg_ref[0, 0] = tg

        @pl.when(k == NBLK)
        def _():
            logz = jnp.sum(jax.lax.slice(part, (0, END), (BAT, END + 1)))
            out_ref[0, 0] = (logz - tg) / float(BAT)


def kernel(feats, target, mask, W, b):
    feats2 = feats.reshape(SEQ * BAT, HID)
    wt = W.T.astype(jnp.bfloat16)
    b2 = b.reshape(1, TT)
    tgt = target[..., 0].astype(jnp.int32).reshape(NBLK, BS, BAT).transpose(0, 2, 1)
    msk = mask.astype(jnp.float32).reshape(NBLK, BS, BAT).transpose(0, 2, 1)
    jj = jnp.arange(TT, dtype=jnp.int32)
    e_mat = (jj[None, :] // T == jnp.arange(T, dtype=jnp.int32)[:, None]).astype(jnp.float32)
    s_mat = (jj[:, None] % T == jnp.arange(T, dtype=jnp.int32)[None, :]).astype(jnp.float32)

    out = pl.pallas_call(
        _crf_body,
        grid=(NBLK + 1,),
        in_specs=[
            pl.BlockSpec((ROWS, HID), lambda k: (jnp.minimum(k, NBLK - 1), 0)),
            pl.BlockSpec((1, BAT, BS), lambda k: (jnp.maximum(k - 1, 0), 0, 0)),
            pl.BlockSpec((1, BAT, BS), lambda k: (jnp.maximum(k - 1, 0), 0, 0)),
            pl.BlockSpec((HID, TT), lambda k: (0, 0)),
            pl.BlockSpec((1, TT), lambda k: (0, 0)),
            pl.BlockSpec((T, TT), lambda k: (0, 0)),
            pl.BlockSpec((TT, T), lambda k: (0, 0)),
        ],
        out_specs=pl.BlockSpec((1, 1), lambda k: (0, 0), memory_space=pltpu.SMEM),
        out_shape=jax.ShapeDtypeStruct((1, 1), jnp.float32),
        scratch_shapes=[
            pltpu.VMEM((2 * ROWS, TT), jnp.float32),
            pltpu.VMEM((BAT, T), jnp.float32),
            pltpu.SMEM((1, 1), jnp.float32),
        ],
        compiler_params=pltpu.CompilerParams(dimension_semantics=("arbitrary",)),
    )(feats2, tgt, msk, wt, b2, e_mat, s_mat)
    return out[0, 0]


# speedup vs baseline: 1.0196x; 1.0196x over previous
import jax
import jax.numpy as jnp
from jax.experimental import pallas as pl
from jax.experimental.pallas import tpu as pltpu

SEQ = 512
BAT = 64
HID = 768
T = 32
TT = T * T
START = 30
END = 31
BS = 8
NBLK = SEQ // BS
ROWS = BS * BAT


def _phase(k, feats_ref, tgt_ref, msk_ref, wt_ref, b_ref, e_ref, s_ref,
           prod_scr, cons_scr, part_ref, tg_ref):
    fb = feats_ref[...].astype(jnp.bfloat16)
    prod_scr[...] = (jnp.dot(fb, wt_ref[...], preferred_element_type=jnp.float32)
                     + b_ref[...])
    lane = jax.lax.broadcasted_iota(jnp.int32, (BAT, TT), 1)
    tgt2 = tgt_ref[0]
    msk2 = msk_ref[0]
    part = part_ref[...]
    tg = jnp.where(k == 1, 0.0, tg_ref[0, 0])
    for i in range(BS):
        sc = cons_scr[i * BAT:(i + 1) * BAT, :]
        tcol = jax.lax.slice(tgt2, (0, i), (BAT, i + 1))
        mcol = jax.lax.slice(msk2, (0, i), (BAT, i + 1))
        tg = tg + jnp.sum(jnp.where((lane == tcol) & (mcol > 0.0), sc, 0.0))
        pmax = jnp.max(part, axis=1, keepdims=True)
        pexp = jnp.dot(part - pmax, e_ref[...],
                       preferred_element_type=jnp.float32)
        cur = sc + pexp
        mrow = jnp.max(cur, axis=1, keepdims=True)
        ex = jnp.exp(cur - mrow)
        ssum = jnp.dot(ex, s_ref[...], preferred_element_type=jnp.float32)
        rec = jnp.log(ssum) + (mrow + pmax)
        newpart = jnp.where(mcol > 0.0, rec, part)
        if i == 0:
            init = jax.lax.slice(sc, (0, START * T), (BAT, START * T + T))
            newpart = jnp.where(k == 1, init, newpart)
        part = newpart
    part_ref[...] = part
    tg_ref[0, 0] = tg


def _crf_body(feats_ref, tgt_ref, msk_ref, wt_ref, b_ref, e_ref, s_ref,
              out_ref, sc_a, sc_b, part_ref, tg_ref):
    k = pl.program_id(0)
    p = jax.lax.rem(k, 2)

    @pl.when(p == 0)
    def _():
        _phase(k, feats_ref, tgt_ref, msk_ref, wt_ref, b_ref, e_ref, s_ref,
               sc_a, sc_b, part_ref, tg_ref)

    @pl.when(p == 1)
    def _():
        _phase(k, feats_ref, tgt_ref, msk_ref, wt_ref, b_ref, e_ref, s_ref,
               sc_b, sc_a, part_ref, tg_ref)

    @pl.when(k == NBLK)
    def _():
        part = part_ref[...]
        logz = jnp.sum(jax.lax.slice(part, (0, END), (BAT, END + 1)))
        out_ref[0, 0] = (logz - tg_ref[0, 0]) / float(BAT)


def kernel(feats, target, mask, W, b):
    feats2 = feats.reshape(SEQ * BAT, HID)
    wt = W.T.astype(jnp.bfloat16)
    b2 = b.reshape(1, TT)
    tgt = target[..., 0].astype(jnp.int32).reshape(NBLK, BS, BAT).transpose(0, 2, 1)
    msk = mask.astype(jnp.float32).reshape(NBLK, BS, BAT).transpose(0, 2, 1)
    jj = jnp.arange(TT, dtype=jnp.int32)
    e_mat = (jj[None, :] // T == jnp.arange(T, dtype=jnp.int32)[:, None]).astype(jnp.float32)
    s_mat = (jj[:, None] % T == jnp.arange(T, dtype=jnp.int32)[None, :]).astype(jnp.float32)

    out = pl.pallas_call(
        _crf_body,
        grid=(NBLK + 1,),
        in_specs=[
            pl.BlockSpec((ROWS, HID), lambda k: (jnp.minimum(k, NBLK - 1), 0)),
            pl.BlockSpec((1, BAT, BS), lambda k: (jnp.maximum(k - 1, 0), 0, 0)),
            pl.BlockSpec((1, BAT, BS), lambda k: (jnp.maximum(k - 1, 0), 0, 0)),
            pl.BlockSpec((HID, TT), lambda k: (0, 0)),
            pl.BlockSpec((1, TT), lambda k: (0, 0)),
            pl.BlockSpec((T, TT), lambda k: (0, 0)),
            pl.BlockSpec((TT, T), lambda k: (0, 0)),
        ],
        out_specs=pl.BlockSpec((1, 1), lambda k: (0, 0), memory_space=pltpu.SMEM),
        out_shape=jax.ShapeDtypeStruct((1, 1), jnp.float32),
        scratch_shapes=[
            pltpu.VMEM((ROWS, TT), jnp.float32),
            pltpu.VMEM((ROWS, TT), jnp.float32),
            pltpu.VMEM((BAT, T), jnp.float32),
            pltpu.SMEM((1, 1), jnp.float32),
        ],
        compiler_params=pltpu.CompilerParams(dimension_semantics=("arbitrary",)),
    )(feats2, tgt, msk, wt, b2, e_mat, s_mat)
    return out[0, 0]


# speedup vs baseline: 1.4068x; 1.3798x over previous
import jax
import jax.numpy as jnp
from jax.experimental import pallas as pl
from jax.experimental.pallas import tpu as pltpu

SEQ = 512
BAT = 64
HID = 768
T = 32
TT = T * T
START = 30
END = 31
BS = 8
NBLK = SEQ // BS
ROWS = BS * BAT
LOG_TT = 6.931471805599453


def _phase(k, feats_ref, tgt_ref, msk_ref, wt_ref, b_ref, e_ref, s_ref,
           prod_scr, cons_scr, q_ref, o_ref, tg_ref):
    fb = feats_ref[...].astype(jnp.bfloat16)
    prod_scr[...] = (jnp.dot(fb, wt_ref[...], preferred_element_type=jnp.float32)
                     + b_ref[...])

    lane = jax.lax.broadcasted_iota(jnp.int32, (BAT, TT), 1)
    tgt2 = tgt_ref[0]
    msk2 = msk_ref[0]
    tg = jnp.where(k == 1, 0.0, tg_ref[0, 0])
    scb, bounds, mcols = [], [], []
    for i in range(BS):
        sc = cons_scr[i * BAT:(i + 1) * BAT, :]
        tcol = jax.lax.slice(tgt2, (0, i), (BAT, i + 1))
        mcol = jax.lax.slice(msk2, (0, i), (BAT, i + 1))
        tg = tg + jnp.sum(jnp.where((lane == tcol) & (mcol > 0.0), sc, 0.0))
        bound = jnp.max(sc, axis=1, keepdims=True) + LOG_TT
        scb.append(sc - bound)
        bounds.append(bound)
        mcols.append(mcol > 0.0)
    tg_ref[0, 0] = tg

    q = q_ref[...]
    o = o_ref[...]
    for i in range(BS):
        pexp = jnp.dot(q, e_ref[...], preferred_element_type=jnp.float32)
        ex = jnp.exp(scb[i] + pexp)
        ssum = jnp.dot(ex, s_ref[...], preferred_element_type=jnp.float32)
        qn = jnp.where(mcols[i], jnp.log(ssum), q)
        on = jnp.where(mcols[i], o + bounds[i], o)
        if i == 0:
            init_q = jax.lax.slice(scb[0], (0, START * T), (BAT, START * T + T))
            qn = jnp.where(k == 1, init_q, qn)
            on = jnp.where(k == 1, jnp.zeros_like(on) + bounds[0], on)
        q, o = qn, on
    q_ref[...] = q
    o_ref[...] = o


def _crf_body(feats_ref, tgt_ref, msk_ref, wt_ref, b_ref, e_ref, s_ref,
              out_ref, sc_a, sc_b, q_ref, o_ref, tg_ref):
    k = pl.program_id(0)
    p = jax.lax.rem(k, 2)

    @pl.when(p == 0)
    def _():
        _phase(k, feats_ref, tgt_ref, msk_ref, wt_ref, b_ref, e_ref, s_ref,
               sc_a, sc_b, q_ref, o_ref, tg_ref)

    @pl.when(p == 1)
    def _():
        _phase(k, feats_ref, tgt_ref, msk_ref, wt_ref, b_ref, e_ref, s_ref,
               sc_b, sc_a, q_ref, o_ref, tg_ref)

    @pl.when(k == NBLK)
    def _():
        pend = q_ref[...] + o_ref[...]
        logz = jnp.sum(jax.lax.slice(pend, (0, END), (BAT, END + 1)))
        out_ref[0, 0] = (logz - tg_ref[0, 0]) / float(BAT)


def kernel(feats, target, mask, W, b):
    feats2 = feats.reshape(SEQ * BAT, HID)
    wt = W.T.astype(jnp.bfloat16)
    b2 = b.reshape(1, TT)
    tgt = target[..., 0].astype(jnp.int32).reshape(NBLK, BS, BAT).transpose(0, 2, 1)
    msk = mask.astype(jnp.float32).reshape(NBLK, BS, BAT).transpose(0, 2, 1)
    jj = jnp.arange(TT, dtype=jnp.int32)
    e_mat = (jj[None, :] // T == jnp.arange(T, dtype=jnp.int32)[:, None]).astype(jnp.float32)
    s_mat = (jj[:, None] % T == jnp.arange(T, dtype=jnp.int32)[None, :]).astype(jnp.float32)

    out = pl.pallas_call(
        _crf_body,
        grid=(NBLK + 1,),
        in_specs=[
            pl.BlockSpec((ROWS, HID), lambda k: (jnp.minimum(k, NBLK - 1), 0)),
            pl.BlockSpec((1, BAT, BS), lambda k: (jnp.maximum(k - 1, 0), 0, 0)),
            pl.BlockSpec((1, BAT, BS), lambda k: (jnp.maximum(k - 1, 0), 0, 0)),
            pl.BlockSpec((HID, TT), lambda k: (0, 0)),
            pl.BlockSpec((1, TT), lambda k: (0, 0)),
            pl.BlockSpec((T, TT), lambda k: (0, 0)),
            pl.BlockSpec((TT, T), lambda k: (0, 0)),
        ],
        out_specs=pl.BlockSpec((1, 1), lambda k: (0, 0), memory_space=pltpu.SMEM),
        out_shape=jax.ShapeDtypeStruct((1, 1), jnp.float32),
        scratch_shapes=[
            pltpu.VMEM((ROWS, TT), jnp.float32),
            pltpu.VMEM((ROWS, TT), jnp.float32),
            pltpu.VMEM((BAT, T), jnp.float32),
            pltpu.VMEM((BAT, T), jnp.float32),
            pltpu.SMEM((1, 1), jnp.float32),
        ],
        compiler_params=pltpu.CompilerParams(dimension_semantics=("arbitrary",)),
    )(feats2, tgt, msk, wt, b2, e_mat, s_mat)
    return out[0, 0]
